# initial kernel scaffold (unmeasured)
import jax
import jax.numpy as jnp
from jax import lax
from jax.experimental import pallas as pl
from jax.experimental.pallas import tpu as pltpu

N_DEV = 4
SQ = 256
SKV_L = 4096
HQ = 8
DH = 128
D = HQ * DH
SCALE = 0.08838834764831843


def kernel(x, Wq, K_ext, V_ext, Wo):
    def body(x_ref, wq_ref, k_ref, v_ref, wo_ref, out_ref,
             o_comm, st_comm, o_send, o_recv, st_send, st_recv):
        my = lax.axis_index("i")
        left = (my - 1) % N_DEV
        right = (my + 1) % N_DEV

        barrier_sem = pltpu.get_barrier_semaphore()
        for nbr in [left, right]:
            pl.semaphore_signal(
                barrier_sem, inc=1,
                device_id=(nbr,), device_id_type=pl.DeviceIdType.MESH,
            )
        pl.semaphore_wait(barrier_sem, 2)

        q = jnp.dot(x_ref[0], wq_ref[...], preferred_element_type=jnp.float32)

        qb = lax.broadcasted_iota(jnp.int32, (SQ, SKV_L), 0) // 64
        kbm = (lax.broadcasted_iota(jnp.int32, (SQ, SKV_L), 1) // 64) % 4
        mask = qb == kbm

        for h in range(HQ):
            qh = q[:, h * DH:(h + 1) * DH]
            kh = k_ref[0, :, h, :]
            s = lax.dot_general(
                qh, kh, (((1,), (1,)), ((), ())),
                preferred_element_type=jnp.float32,
            ) * SCALE
            s = jnp.where(mask, s, -1e9)
            m = jnp.max(s, axis=1)
            p = jnp.exp(s - m[:, None])
            l = jnp.sum(p, axis=1)
            o = jnp.dot(p, v_ref[0, :, h, :],
                        preferred_element_type=jnp.float32)
            o_comm[my, :, h * DH:(h + 1) * DH] = o
            st_comm[my, 0, h, :] = m
            st_comm[my, 1, h, :] = l

        for hop in range(N_DEV - 1):
            slot = (my - hop) % N_DEV
            rdma_o = pltpu.make_async_remote_copy(
                src_ref=o_comm.at[slot],
                dst_ref=o_comm.at[slot],
                send_sem=o_send.at[hop],
                recv_sem=o_recv.at[hop],
                device_id=(right,),
                device_id_type=pl.DeviceIdType.MESH,
            )
            rdma_st = pltpu.make_async_remote_copy(
                src_ref=st_comm.at[slot],
                dst_ref=st_comm.at[slot],
                send_sem=st_send.at[hop],
                recv_sem=st_recv.at[hop],
                device_id=(right,),
                device_id_type=pl.DeviceIdType.MESH,
            )
            rdma_o.start()
            rdma_st.start()
            rdma_o.wait()
            rdma_st.wait()

        m_all = st_comm[:, 0, :, :]
        l_all = st_comm[:, 1, :, :]
        m_g = jnp.max(m_all, axis=0)
        alpha = jnp.exp(m_all - m_g[None])
        l_g = jnp.sum(l_all * alpha, axis=0)

        alpha_e = jnp.broadcast_to(
            alpha.transpose(0, 2, 1)[:, :, :, None], (N_DEV, SQ, HQ, DH)
        ).reshape(N_DEV, SQ, D)
        acc = jnp.sum(o_comm[...] * alpha_e, axis=0)

        linv_e = jnp.broadcast_to(
            (1.0 / l_g).transpose(1, 0)[:, :, None], (SQ, HQ, DH)
        ).reshape(SQ, D)
        ctx = acc * linv_e

        out_ref[0] = jnp.dot(ctx, wo_ref[...],
                             preferred_element_type=jnp.float32)

    return pl.pallas_call(
        body,
        out_shape=jax.ShapeDtypeStruct((1, SQ, D), jnp.float32),
        in_specs=[pl.BlockSpec(memory_space=pltpu.VMEM)] * 5,
        out_specs=pl.BlockSpec(memory_space=pltpu.VMEM),
        scratch_shapes=[
            pltpu.VMEM((N_DEV, SQ, D), jnp.float32),
            pltpu.VMEM((N_DEV, 2, HQ, SQ), jnp.float32),
            pltpu.SemaphoreType.DMA((N_DEV - 1,)),
            pltpu.SemaphoreType.DMA((N_DEV - 1,)),
            pltpu.SemaphoreType.DMA((N_DEV - 1,)),
            pltpu.SemaphoreType.DMA((N_DEV - 1,)),
        ],
        compiler_params=pltpu.CompilerParams(collective_id=0),
    )(x, Wq, K_ext, V_ext, Wo)


# baseline (device time: 92688 ns/iter reference)
import jax
import jax.numpy as jnp
from jax import lax
from jax.experimental import pallas as pl
from jax.experimental.pallas import tpu as pltpu

N_DEV = 4
SQ = 256
SKV_L = 4096
HQ = 8
DH = 128
D = HQ * DH
SCALE = 0.08838834764831843


def kernel(x, Wq, K_ext, V_ext, Wo):
    def body(x_ref, wq_ref, k_ref, v_ref, wo_ref, out_ref,
             k_buf, v_buf, o_comm, st_comm, copy_sems,
             o_send, o_recv, st_send, st_recv):
        my = lax.axis_index("i")
        left = (my - 1) % N_DEV
        right = (my + 1) % N_DEV

        barrier_sem = pltpu.get_barrier_semaphore()
        for nbr in [left, right]:
            pl.semaphore_signal(
                barrier_sem, inc=1,
                device_id=(nbr,), device_id_type=pl.DeviceIdType.MESH,
            )
        pl.semaphore_wait(barrier_sem, 2)

        q = jnp.dot(x_ref[0], wq_ref[...], preferred_element_type=jnp.float32)

        qb = lax.broadcasted_iota(jnp.int32, (SQ, SKV_L), 0) // 64
        kbm = (lax.broadcasted_iota(jnp.int32, (SQ, SKV_L), 1) // 64) % 4
        mask = qb == kbm

        for h in range(HQ):
            ck = pltpu.make_async_copy(
                k_ref.at[0, :, h, :], k_buf, copy_sems.at[0])
            cv = pltpu.make_async_copy(
                v_ref.at[0, :, h, :], v_buf, copy_sems.at[1])
            ck.start()
            cv.start()
            ck.wait()
            cv.wait()
            qh = q[:, h * DH:(h + 1) * DH]
            s = lax.dot_general(
                qh, k_buf[...], (((1,), (1,)), ((), ())),
                preferred_element_type=jnp.float32,
            ) * SCALE
            s = jnp.where(mask, s, -1e9)
            m = jnp.max(s, axis=1)
            p = jnp.exp(s - m[:, None])
            l = jnp.sum(p, axis=1)
            o = jnp.dot(p, v_buf[...],
                        preferred_element_type=jnp.float32)
            o_comm[my, :, h * DH:(h + 1) * DH] = o
            st_comm[my, 0, h, :] = m
            st_comm[my, 1, h, :] = l

        for hop in range(N_DEV - 1):
            slot = (my - hop) % N_DEV
            rdma_o = pltpu.make_async_remote_copy(
                src_ref=o_comm.at[slot],
                dst_ref=o_comm.at[slot],
                send_sem=o_send.at[hop],
                recv_sem=o_recv.at[hop],
                device_id=(right,),
                device_id_type=pl.DeviceIdType.MESH,
            )
            rdma_st = pltpu.make_async_remote_copy(
                src_ref=st_comm.at[slot],
                dst_ref=st_comm.at[slot],
                send_sem=st_send.at[hop],
                recv_sem=st_recv.at[hop],
                device_id=(right,),
                device_id_type=pl.DeviceIdType.MESH,
            )
            rdma_o.start()
            rdma_st.start()
            rdma_o.wait()
            rdma_st.wait()

        m_all = st_comm[:, 0, :, :]
        l_all = st_comm[:, 1, :, :]
        m_g = jnp.max(m_all, axis=0)
        alpha = jnp.exp(m_all - m_g[None])
        l_g = jnp.sum(l_all * alpha, axis=0)

        alpha_e = jnp.broadcast_to(
            alpha.transpose(0, 2, 1)[:, :, :, None], (N_DEV, SQ, HQ, DH)
        ).reshape(N_DEV, SQ, D)
        acc = jnp.sum(o_comm[...] * alpha_e, axis=0)

        linv_e = jnp.broadcast_to(
            (1.0 / l_g).transpose(1, 0)[:, :, None], (SQ, HQ, DH)
        ).reshape(SQ, D)
        ctx = acc * linv_e

        out_ref[0] = jnp.dot(ctx, wo_ref[...],
                             preferred_element_type=jnp.float32)

    return pl.pallas_call(
        body,
        out_shape=jax.ShapeDtypeStruct((1, SQ, D), jnp.float32),
        in_specs=[
            pl.BlockSpec(memory_space=pltpu.VMEM),
            pl.BlockSpec(memory_space=pltpu.VMEM),
            pl.BlockSpec(memory_space=pl.ANY),
            pl.BlockSpec(memory_space=pl.ANY),
            pl.BlockSpec(memory_space=pltpu.VMEM),
        ],
        out_specs=pl.BlockSpec(memory_space=pltpu.VMEM),
        scratch_shapes=[
            pltpu.VMEM((SKV_L, DH), jnp.float32),
            pltpu.VMEM((SKV_L, DH), jnp.float32),
            pltpu.VMEM((N_DEV, SQ, D), jnp.float32),
            pltpu.VMEM((N_DEV, 2, HQ, SQ), jnp.float32),
            pltpu.SemaphoreType.DMA((2,)),
            pltpu.SemaphoreType.DMA((N_DEV - 1,)),
            pltpu.SemaphoreType.DMA((N_DEV - 1,)),
            pltpu.SemaphoreType.DMA((N_DEV - 1,)),
            pltpu.SemaphoreType.DMA((N_DEV - 1,)),
        ],
        compiler_params=pltpu.CompilerParams(collective_id=0),
    )(x, Wq, K_ext, V_ext, Wo)


# device time: 88723 ns/iter; 1.0447x vs baseline; 1.0447x over previous
import jax
import jax.numpy as jnp
from jax import lax
from jax.experimental import pallas as pl
from jax.experimental.pallas import tpu as pltpu

N_DEV = 4
SQ = 256
SKV_L = 4096
HQ = 8
DH = 128
D = HQ * DH
SCALE = 0.08838834764831843


def kernel(x, Wq, K_ext, V_ext, Wo):
    NG = 4
    GK = SKV_L // NG
    NB = GK // 64

    def body(x_ref, wq_ref, k_ref, v_ref, wo_ref, out_ref,
             k_buf, v_buf, o_comm, st_comm, copy_sems,
             o_send, o_recv, st_send, st_recv):
        my = lax.axis_index("i")
        left = (my - 1) % N_DEV
        right = (my + 1) % N_DEV

        barrier_sem = pltpu.get_barrier_semaphore()
        for nbr in [left, right]:
            pl.semaphore_signal(
                barrier_sem, inc=1,
                device_id=(nbr,), device_id_type=pl.DeviceIdType.MESH,
            )
        pl.semaphore_wait(barrier_sem, 2)

        q = jnp.dot(x_ref[0], wq_ref[...], preferred_element_type=jnp.float32)

        def start_group_copies(r):
            copies = []
            for j in range(NB):
                b = NG * j + r
                copies.append(pltpu.make_async_copy(
                    k_ref.at[0, pl.ds(b * 64, 64)],
                    k_buf.at[pl.ds(j * 64, 64)],
                    copy_sems.at[0]))
                copies.append(pltpu.make_async_copy(
                    v_ref.at[0, pl.ds(b * 64, 64)],
                    v_buf.at[pl.ds(j * 64, 64)],
                    copy_sems.at[1]))
            for c in copies:
                c.start()
            return copies

        for r in range(NG):
            copies = start_group_copies(r)
            for c in copies:
                c.wait()
            q_r = q[r * 64:(r + 1) * 64, :]
            for h in range(HQ):
                qh = q_r[:, h * DH:(h + 1) * DH]
                s = lax.dot_general(
                    qh, k_buf[:, h, :], (((1,), (1,)), ((), ())),
                    preferred_element_type=jnp.float32,
                ) * SCALE
                m = jnp.max(s, axis=1)
                p = jnp.exp(s - m[:, None])
                l = jnp.sum(p, axis=1)
                o = jnp.dot(p, v_buf[:, h, :],
                            preferred_element_type=jnp.float32)
                o_comm[my, r * 64:(r + 1) * 64, h * DH:(h + 1) * DH] = o
                st_comm[my, 0, h, r * 64:(r + 1) * 64] = m
                st_comm[my, 1, h, r * 64:(r + 1) * 64] = l

        for hop in range(N_DEV - 1):
            slot = (my - hop) % N_DEV
            rdma_o = pltpu.make_async_remote_copy(
                src_ref=o_comm.at[slot],
                dst_ref=o_comm.at[slot],
                send_sem=o_send.at[hop],
                recv_sem=o_recv.at[hop],
                device_id=(right,),
                device_id_type=pl.DeviceIdType.MESH,
            )
            rdma_st = pltpu.make_async_remote_copy(
                src_ref=st_comm.at[slot],
                dst_ref=st_comm.at[slot],
                send_sem=st_send.at[hop],
                recv_sem=st_recv.at[hop],
                device_id=(right,),
                device_id_type=pl.DeviceIdType.MESH,
            )
            rdma_o.start()
            rdma_st.start()
            rdma_o.wait()
            rdma_st.wait()

        m_all = st_comm[:, 0, :, :]
        l_all = st_comm[:, 1, :, :]
        m_g = jnp.max(m_all, axis=0)
        alpha = jnp.exp(m_all - m_g[None])
        l_g = jnp.sum(l_all * alpha, axis=0)

        alpha_e = jnp.broadcast_to(
            alpha.transpose(0, 2, 1)[:, :, :, None], (N_DEV, SQ, HQ, DH)
        ).reshape(N_DEV, SQ, D)
        acc = jnp.sum(o_comm[...] * alpha_e, axis=0)

        linv_e = jnp.broadcast_to(
            (1.0 / l_g).transpose(1, 0)[:, :, None], (SQ, HQ, DH)
        ).reshape(SQ, D)
        ctx = acc * linv_e

        out_ref[0] = jnp.dot(ctx, wo_ref[...],
                             preferred_element_type=jnp.float32)

    return pl.pallas_call(
        body,
        out_shape=jax.ShapeDtypeStruct((1, SQ, D), jnp.float32),
        in_specs=[
            pl.BlockSpec(memory_space=pltpu.VMEM),
            pl.BlockSpec(memory_space=pltpu.VMEM),
            pl.BlockSpec(memory_space=pl.ANY),
            pl.BlockSpec(memory_space=pl.ANY),
            pl.BlockSpec(memory_space=pltpu.VMEM),
        ],
        out_specs=pl.BlockSpec(memory_space=pltpu.VMEM),
        scratch_shapes=[
            pltpu.VMEM((SKV_L // 4, HQ, DH), jnp.float32),
            pltpu.VMEM((SKV_L // 4, HQ, DH), jnp.float32),
            pltpu.VMEM((N_DEV, SQ, D), jnp.float32),
            pltpu.VMEM((N_DEV, 2, HQ, SQ), jnp.float32),
            pltpu.SemaphoreType.DMA((2,)),
            pltpu.SemaphoreType.DMA((N_DEV - 1,)),
            pltpu.SemaphoreType.DMA((N_DEV - 1,)),
            pltpu.SemaphoreType.DMA((N_DEV - 1,)),
            pltpu.SemaphoreType.DMA((N_DEV - 1,)),
        ],
        compiler_params=pltpu.CompilerParams(collective_id=0),
    )(x, Wq, K_ext, V_ext, Wo)


# device time: 44910 ns/iter; 2.0639x vs baseline; 1.9756x over previous
import jax
import jax.numpy as jnp
from jax import lax
from jax.experimental import pallas as pl
from jax.experimental.pallas import tpu as pltpu

N_DEV = 4
SQ = 256
SKV_L = 4096
HQ = 8
DH = 128
D = HQ * DH
SCALE = 0.08838834764831843
COMM = True


def kernel(x, Wq, K_ext, V_ext, Wo):
    NG = 4
    GK = SKV_L // NG
    NB = GK // 64

    def body(x_ref, wq_ref, k_ref, v_ref, wo_ref, out_ref,
             k_buf, v_buf, o_comm, st_comm, ctx_buf, copy_sems,
             o_send, o_recv, st_send, st_recv, opp_credit):
        my = lax.axis_index("i")
        left = (my - 1) % N_DEV
        right = (my + 1) % N_DEV
        opp = (my + 2) % N_DEV

        def start_group_copies(r, buf):
            per_head = []
            for h in range(HQ):
                copies = []
                for j in range(NB):
                    b = NG * j + r
                    copies.append(pltpu.make_async_copy(
                        k_ref.at[0, pl.ds(b * 64, 64), h, :],
                        k_buf.at[buf, h, pl.ds(j * 64, 64), :],
                        copy_sems.at[buf, 0, h]))
                    copies.append(pltpu.make_async_copy(
                        v_ref.at[0, pl.ds(b * 64, 64), h, :],
                        v_buf.at[buf, h, pl.ds(j * 64, 64), :],
                        copy_sems.at[buf, 1, h]))
                for c in copies:
                    c.start()
                per_head.append(copies)
            return per_head

        all_copies = [start_group_copies(r, r) for r in range(NG)]

        barrier_sem = pltpu.get_barrier_semaphore()
        for nbr in [left, right]:
            pl.semaphore_signal(
                barrier_sem, inc=1,
                device_id=(nbr,), device_id_type=pl.DeviceIdType.MESH,
            )
        pl.semaphore_wait(barrier_sem, 2)

        q = jnp.dot(x_ref[0], wq_ref[...], preferred_element_type=jnp.float32)

        RH = ((0, right), (1, left), (2, opp))
        HALF = D // 2
        a_desc = [[None] * NG for _ in range(3)]
        a3 = [[None] * 3, [None] * 3]

        def start_a3(half):
            cols = pl.ds(half * HALF, HALF)
            rows = pl.ds((NG - 1) * 64, 64)
            for dirn, nbr in (RH if COMM else ()):
                d = pltpu.make_async_remote_copy(
                    src_ref=o_comm.at[my, rows, cols],
                    dst_ref=o_comm.at[my, rows, cols],
                    send_sem=o_send.at[dirn, NG - 1 + half],
                    recv_sem=o_recv.at[dirn, NG - 1 + half],
                    device_id=(nbr,),
                    device_id_type=pl.DeviceIdType.MESH,
                )
                d.start()
                a3[half][dirn] = d

        for r in range(NG):
            buf = r
            copies = all_copies[r]
            q_r = q[r * 64:(r + 1) * 64, :]
            for h in range(HQ):
                for c in copies[h]:
                    c.wait()
                qh = q_r[:, h * DH:(h + 1) * DH]
                s = lax.dot_general(
                    qh, k_buf[buf, h], (((1,), (1,)), ((), ())),
                    preferred_element_type=jnp.float32,
                ) * SCALE
                m = jnp.max(s, axis=1)
                p = jnp.exp(s - m[:, None])
                l = jnp.sum(p, axis=1)
                o = jnp.dot(p, v_buf[buf, h],
                            preferred_element_type=jnp.float32)
                o_comm[my, r * 64:(r + 1) * 64, h * DH:(h + 1) * DH] = (
                    o.astype(jnp.bfloat16))
                st_comm[my, 0, h, r * 64:(r + 1) * 64] = m
                st_comm[my, 1, h, r * 64:(r + 1) * 64] = l
                if r == NG - 1 and h == HQ // 2 - 1:
                    start_a3(0)
                if r == NG - 1 and h == HQ - 1:
                    start_a3(1)
            for dirn, nbr in (RH if (COMM and r < NG - 1) else ()):
                d = pltpu.make_async_remote_copy(
                    src_ref=o_comm.at[my, pl.ds(r * 64, 64)],
                    dst_ref=o_comm.at[my, pl.ds(r * 64, 64)],
                    send_sem=o_send.at[dirn, r],
                    recv_sem=o_recv.at[dirn, r],
                    device_id=(nbr,),
                    device_id_type=pl.DeviceIdType.MESH,
                )
                d.start()
                a_desc[dirn][r] = d

        st_a = []
        for dirn, nbr in (RH if COMM else ()):
            d = pltpu.make_async_remote_copy(
                src_ref=st_comm.at[my],
                dst_ref=st_comm.at[my],
                send_sem=st_send.at[dirn],
                recv_sem=st_recv.at[dirn],
                device_id=(nbr,),
                device_id_type=pl.DeviceIdType.MESH,
            )
            d.start()
            st_a.append(d)

        for d in st_a:
            d.wait_recv()

        m_all = st_comm[:, 0, :, :]
        l_all = st_comm[:, 1, :, :]
        m_g = jnp.max(m_all, axis=0)
        alpha = jnp.exp(m_all - m_g[None])
        l_g = jnp.sum(l_all * alpha, axis=0)
        scale = alpha / l_g[None]
        scale_t = scale.transpose(0, 2, 1)

        for r in range(NG):
            if COMM:
                if r < NG - 1:
                    for dirn in (0, 1, 2):
                        a_desc[dirn][r].wait_recv()
                else:
                    for half in (0, 1):
                        for dirn in (0, 1, 2):
                            a3[half][dirn].wait_recv()
            rows = pl.ds(r * 64, 64)
            sc_r = jnp.broadcast_to(
                scale_t[:, r * 64:(r + 1) * 64, :, None],
                (N_DEV, 64, HQ, DH)).reshape(N_DEV, 64, D)
            o_r = o_comm[:, rows, :].astype(jnp.float32)
            ctx_buf[rows, :] = jnp.sum(o_r * sc_r, axis=0)

        out_ref[0] = jnp.dot(ctx_buf[...], wo_ref[...],
                             preferred_element_type=jnp.float32)

        if COMM:
            pl.semaphore_signal(
                opp_credit, inc=1,
                device_id=(opp,), device_id_type=pl.DeviceIdType.MESH,
            )
            pl.semaphore_wait(opp_credit, 1)

        for dirn in (0, 1, 2):
            for r in (range(NG) if COMM else ()):
                if r < NG - 1:
                    a_desc[dirn][r].wait_send()
        if COMM:
            for half in (0, 1):
                for dirn in (0, 1, 2):
                    a3[half][dirn].wait_send()
        for d in st_a:
            d.wait_send()

    return pl.pallas_call(
        body,
        out_shape=jax.ShapeDtypeStruct((1, SQ, D), jnp.float32),
        in_specs=[
            pl.BlockSpec(memory_space=pltpu.VMEM),
            pl.BlockSpec(memory_space=pltpu.VMEM),
            pl.BlockSpec(memory_space=pl.ANY),
            pl.BlockSpec(memory_space=pl.ANY),
            pl.BlockSpec(memory_space=pltpu.VMEM),
        ],
        out_specs=pl.BlockSpec(memory_space=pltpu.VMEM),
        scratch_shapes=[
            pltpu.VMEM((4, HQ, SKV_L // 4, DH), jnp.float32),
            pltpu.VMEM((4, HQ, SKV_L // 4, DH), jnp.float32),
            pltpu.VMEM((N_DEV, SQ, D), jnp.bfloat16),
            pltpu.VMEM((N_DEV, 2, HQ, SQ), jnp.float32),
            pltpu.VMEM((SQ, D), jnp.float32),
            pltpu.SemaphoreType.DMA((4, 2, HQ)),
            pltpu.SemaphoreType.DMA((3, 5)),
            pltpu.SemaphoreType.DMA((3, 5)),
            pltpu.SemaphoreType.DMA((3,)),
            pltpu.SemaphoreType.DMA((3,)),
            pltpu.SemaphoreType.REGULAR,
        ],
        compiler_params=pltpu.CompilerParams(
            collective_id=0, vmem_limit_bytes=60 * 1024 * 1024),
    )(x, Wq, K_ext, V_ext, Wo)
